# Initial kernel scaffold; baseline (speedup 1.0000x reference)
import jax
import jax.numpy as jnp
from jax.experimental import pallas as pl


def kernel(gat_w, gat_al, gat_ar, gat_bias, sem_w1, sem_b1, sem_w2, pred_w, pred_b, mlp_w1, mlp_b1, mlp_w2, mlp_b2, mlp_w3, mlp_b3, adj, h):
    raise NotImplementedError("write your pallas kernel here")


# speedup vs baseline: 2.2974x; 2.2974x over previous
import functools

import jax
import jax.numpy as jnp
from jax import lax
from jax.experimental import pallas as pl
from jax.experimental.pallas import tpu as pltpu

VMEM = pltpu.MemorySpace.VMEM


def _gat_kernel(adj_ref, h_ref, w_ref, al_ref, ar_ref, b_ref,
                sw1_ref, sb1_ref, sw2_ref, o_ref,
                feat_ref, bd_ref, cl_ref, f0_ref, f1_ref,
                *, num_heads, head_dim, tile_m, bph, npg):
    hf = pl.program_id(1)
    r = pl.program_id(2)
    n = h_ref.shape[0]
    d = num_heads * head_dim

    @pl.when(r == 0)
    def _():
        feat = jnp.dot(h_ref[...].astype(jnp.bfloat16),
                       w_ref[...].astype(jnp.bfloat16),
                       preferred_element_type=jnp.float32)
        feat_ref[...] = feat
        feat_bf = feat.astype(jnp.bfloat16)
        ones = jnp.ones((n, 1), jnp.bfloat16)
        zer = jnp.zeros((n, head_dim - 1), jnp.bfloat16)
        for hd in range(num_heads):
            lo = hd * head_dim
            fh = feat[:, lo:lo + head_dim]
            al = al_ref[hd:hd + 1, :]
            el = lax.dot_general(al, fh, (((1,), (1,)), ((), ())),
                                 preferred_element_type=jnp.float32)
            cl = jnp.max(el, axis=1, keepdims=True)
            cl_ref[hd:hd + 1, 0:1] = cl
            bd_ref[2 * hd:2 * hd + 1, :] = jnp.exp(el - cl).astype(jnp.bfloat16)
            bd_ref[2 * hd + 1:2 * hd + 2, :] = (
                jnp.exp(0.2 * (el - cl)).astype(jnp.bfloat16))
            faug = jnp.concatenate(
                [feat_bf[:, lo:lo + head_dim], ones, zer], axis=1)
            if hd == 0:
                f0_ref[...] = faug
            else:
                f1_ref[...] = faug

    row0 = (hf * bph + r) * tile_m
    mask_bf = adj_ref[...].astype(jnp.bfloat16)
    feat_dst = feat_ref[pl.ds(row0, tile_m), :]
    bias = b_ref[...]

    outs = []
    for hd in range(num_heads):
        lo = hd * head_dim
        fd = feat_dst[:, lo:lo + head_dim]
        ar = ar_ref[hd:hd + 1, :]
        er = jnp.sum(fd * ar, axis=1, keepdims=True)
        u = er + cl_ref[hd:hd + 1, 0:1]
        a_dst = jnp.exp(0.8 * jnp.minimum(u, 0.0)).astype(jnp.bfloat16)
        c_dst = jnp.exp(-0.8 * jnp.maximum(u, 0.0)).astype(jnp.bfloat16)
        b_src = bd_ref[2 * hd:2 * hd + 1, :]
        d_src = bd_ref[2 * hd + 1:2 * hd + 2, :]

        p = jnp.maximum(a_dst * b_src, c_dst * d_src) * mask_bf
        fa = f0_ref[...] if hd == 0 else f1_ref[...]
        mm = jnp.dot(p, fa, preferred_element_type=jnp.float32)
        num = mm[:, :head_dim]
        den = jnp.maximum(mm[:, head_dim:head_dim + 1], 1e-30)
        out_h = num / den + bias[0:1, lo:lo + head_dim]
        out_h = jnp.where(out_h > 0, out_h,
                          jnp.exp(jnp.minimum(out_h, 0.0)) - 1.0)
        outs.append(out_h)

    o = jnp.concatenate(outs, axis=1)

    gpt = tile_m // npg
    g = jnp.sum(o.reshape(gpt, npg, d), axis=1)

    t = jnp.tanh(jnp.dot(o, sw1_ref[...],
                         preferred_element_type=jnp.float32) + sb1_ref[...])
    sc = jnp.dot(t, sw2_ref[...], preferred_element_type=jnp.float32)
    spart = jnp.full((1, d), jnp.sum(sc), jnp.float32)

    pad = jnp.zeros((8 - gpt - 1, d), jnp.float32)
    o_ref[...] = jnp.concatenate([g, spart, pad], axis=0)[None]


def _gat_all(adj, h, w, al, ar, bias, sw1, sb1, sw2,
             *, num_heads, head_dim, tile_m, npg):
    m, n, _ = adj.shape
    fin = h.shape[-1]
    d = num_heads * head_dim
    tiles = n // tile_m
    bph = tiles // 2

    flops = int(m * (2 * n * fin * d
                     + num_heads * (2 * n * head_dim + 2 * n * n * head_dim)
                     + 2 * n * d * 128))
    transcendentals = int(m * n * 130)
    bytes_accessed = int(m * (n * n * adj.dtype.itemsize + n * fin * 4))

    body = functools.partial(_gat_kernel, num_heads=num_heads,
                             head_dim=head_dim, tile_m=tile_m, bph=bph,
                             npg=npg)
    return pl.pallas_call(
        body,
        out_shape=jax.ShapeDtypeStruct((m, tiles, 8, d), jnp.float32),
        grid=(m, 2, bph),
        in_specs=[
            pl.BlockSpec((None, tile_m, n),
                         lambda i, hf, r: (i, hf * bph + r, 0)),
            pl.BlockSpec((None, n, fin), lambda i, hf, r: (i, 0, 0)),
            pl.BlockSpec((None, fin, d), lambda i, hf, r: (i, 0, 0)),
            pl.BlockSpec((None, num_heads, head_dim),
                         lambda i, hf, r: (i, 0, 0)),
            pl.BlockSpec((None, num_heads, head_dim),
                         lambda i, hf, r: (i, 0, 0)),
            pl.BlockSpec((None, 1, d), lambda i, hf, r: (i, 0, 0)),
            pl.BlockSpec(sw1.shape, lambda i, hf, r: (0, 0)),
            pl.BlockSpec(sb1.shape, lambda i, hf, r: (0, 0)),
            pl.BlockSpec(sw2.shape, lambda i, hf, r: (0, 0)),
        ],
        out_specs=pl.BlockSpec((None, 1, 8, d),
                               lambda i, hf, r: (i, hf * bph + r, 0, 0)),
        scratch_shapes=[
            pltpu.VMEM((n, d), jnp.float32),
            pltpu.VMEM((8, n), jnp.bfloat16),
            pltpu.VMEM((8, 128), jnp.float32),
            pltpu.VMEM((n, 2 * head_dim), jnp.bfloat16),
            pltpu.VMEM((n, 2 * head_dim), jnp.bfloat16),
        ],
        compiler_params=pltpu.CompilerParams(
            dimension_semantics=("parallel", "parallel", "arbitrary"),
            vmem_limit_bytes=64 * 1024 * 1024),
        cost_estimate=pl.CostEstimate(flops=flops,
                                      transcendentals=transcendentals,
                                      bytes_accessed=bytes_accessed),
    )(adj, h, w, al, ar, bias, sw1, sb1, sw2)


def _head_kernel(ga_ref, pw_ref, pb_ref, mw1_ref, mb1_ref, mw2_ref, mb2_ref,
                 mw3_ref, mb3_ref, o_ref, *, n_nodes, npg, tile_m):
    ga = ga_ref[...]
    mcount, tiles, _, d = ga.shape
    gpt = tile_m // npg

    sc = ga[:, :, gpt:gpt + 1, 0:1]
    scores = jnp.sum(sc, axis=1, keepdims=True) * (1.0 / n_nodes)
    mx = jnp.max(scores, axis=0, keepdims=True)
    e = jnp.exp(scores - mx)
    beta = e / jnp.sum(e, axis=0, keepdims=True)

    gsum = jnp.sum(ga[:, :, 0:gpt, :] * beta, axis=0)
    gm = gsum.reshape(tiles * gpt, d)

    g = (jnp.dot(gm, pw_ref[...], preferred_element_type=jnp.float32)
         + npg * pb_ref[...])
    x = jnp.maximum(jnp.dot(g, mw1_ref[...],
                            preferred_element_type=jnp.float32)
                    + mb1_ref[...], 0.0)
    x = jnp.maximum(jnp.dot(x, mw2_ref[...],
                            preferred_element_type=jnp.float32)
                    + mb2_ref[...], 0.0)
    logits = jnp.dot(x, mw3_ref[...],
                     preferred_element_type=jnp.float32) + mb3_ref[...]
    mmax = jnp.max(logits, axis=1, keepdims=True)
    p = jnp.exp(logits - mmax)
    o_ref[...] = p / jnp.sum(p, axis=1, keepdims=True)


def _head(ga, pred_w, pred_b, mlp, *, batch, npg, tile_m):
    n_nodes = batch * npg
    body = functools.partial(_head_kernel, n_nodes=n_nodes, npg=npg,
                             tile_m=tile_m)
    vspec = pl.BlockSpec(memory_space=VMEM)
    return pl.pallas_call(
        body,
        out_shape=jax.ShapeDtypeStruct((batch, 2), jnp.float32),
        in_specs=[vspec] * 9,
        out_specs=vspec,
    )(ga, pred_w, pred_b, mlp["w1"], mlp["b1"], mlp["w2"], mlp["b2"],
      mlp["w3"], mlp["b3"])


def _han(gat_w, gat_al, gat_ar, gat_bias, sem_w1, sem_b1, sem_w2,
         pred_w, pred_b, mlp, adj, h, *, num_heads, head_dim, batch, npg,
         tile_m):
    ga = _gat_all(adj, h, gat_w, gat_al, gat_ar, gat_bias,
                  sem_w1, sem_b1, sem_w2,
                  num_heads=num_heads, head_dim=head_dim, tile_m=tile_m,
                  npg=npg)
    return _head(ga, pred_w, pred_b, mlp, batch=batch, npg=npg, tile_m=tile_m)


def kernel(gat_w, gat_al, gat_ar, gat_bias, sem_w1, sem_b1, sem_w2,
           pred_w, pred_b, mlp_w1, mlp_b1, mlp_w2, mlp_b2, mlp_w3, mlp_b3,
           adj, h):
    mlp = {"w1": mlp_w1, "b1": mlp_b1, "w2": mlp_w2, "b2": mlp_b2,
           "w3": mlp_w3, "b3": mlp_b3}
    return _han(gat_w, gat_al, gat_ar, gat_bias, sem_w1, sem_b1, sem_w2,
                pred_w, pred_b, mlp, adj, h,
                num_heads=2, head_dim=8, batch=64, npg=64, tile_m=256)


# speedup vs baseline: 2.4162x; 1.0517x over previous
import functools

import jax
import jax.numpy as jnp
from jax import lax
from jax.experimental import pallas as pl
from jax.experimental.pallas import tpu as pltpu

VMEM = pltpu.MemorySpace.VMEM


def _gat_kernel(adj_ref, h_ref, w_ref, al_ref, ar_ref, b_ref,
                sw1_ref, sb1_ref, sw2_ref, o_ref,
                feat_ref, bd_ref, cl_ref, f0_ref, f1_ref,
                *, num_heads, head_dim, tile_m, bph, npg):
    hf = pl.program_id(1)
    r = pl.program_id(2)
    n = h_ref.shape[0]
    d = num_heads * head_dim

    @pl.when(r == 0)
    def _():
        feat = jnp.dot(h_ref[...].astype(jnp.bfloat16),
                       w_ref[...].astype(jnp.bfloat16),
                       preferred_element_type=jnp.float32)
        feat_ref[...] = feat
        feat_bf = feat.astype(jnp.bfloat16)
        ones = jnp.ones((n, 1), jnp.bfloat16)
        zer = jnp.zeros((n, head_dim - 1), jnp.bfloat16)
        for hd in range(num_heads):
            lo = hd * head_dim
            fh = feat[:, lo:lo + head_dim]
            al = al_ref[hd:hd + 1, :]
            el = lax.dot_general(al, fh, (((1,), (1,)), ((), ())),
                                 preferred_element_type=jnp.float32)
            cl = jnp.max(el, axis=1, keepdims=True)
            cl_ref[hd:hd + 1, 0:1] = cl
            bd_ref[2 * hd:2 * hd + 1, :] = jnp.exp(el - cl).astype(jnp.bfloat16)
            bd_ref[2 * hd + 1:2 * hd + 2, :] = (
                jnp.exp(0.2 * (el - cl)).astype(jnp.bfloat16))
            faug = jnp.concatenate(
                [feat_bf[:, lo:lo + head_dim], ones, zer], axis=1)
            if hd == 0:
                f0_ref[...] = faug
            else:
                f1_ref[...] = faug

    row0 = (hf * bph + r) * tile_m
    mask_bf = adj_ref[...].astype(jnp.bfloat16)
    feat_dst = feat_ref[pl.ds(row0, tile_m), :]
    bias = b_ref[...]

    outs = []
    for hd in range(num_heads):
        lo = hd * head_dim
        fd = feat_dst[:, lo:lo + head_dim]
        ar = ar_ref[hd:hd + 1, :]
        er = jnp.sum(fd * ar, axis=1, keepdims=True)
        u = er + cl_ref[hd:hd + 1, 0:1]
        a_dst = jnp.exp(0.8 * jnp.minimum(u, 0.0)).astype(jnp.bfloat16)
        c_dst = jnp.exp(-0.8 * jnp.maximum(u, 0.0)).astype(jnp.bfloat16)
        b_src = bd_ref[2 * hd:2 * hd + 1, :]
        d_src = bd_ref[2 * hd + 1:2 * hd + 2, :]

        p = jnp.maximum(a_dst * b_src, c_dst * d_src) * mask_bf
        fa = f0_ref[...] if hd == 0 else f1_ref[...]
        mm = jnp.dot(p, fa, preferred_element_type=jnp.float32)
        num = mm[:, :head_dim]
        den = jnp.maximum(mm[:, head_dim:head_dim + 1], 1e-30)
        out_h = num / den + bias[0:1, lo:lo + head_dim]
        out_h = jnp.where(out_h > 0, out_h,
                          jnp.exp(jnp.minimum(out_h, 0.0)) - 1.0)
        outs.append(out_h)

    o = jnp.concatenate(outs, axis=1)

    gpt = tile_m // npg
    g = jnp.sum(o.reshape(gpt, npg, d), axis=1)

    t = jnp.tanh(jnp.dot(o, sw1_ref[...],
                         preferred_element_type=jnp.float32) + sb1_ref[...])
    sc = jnp.dot(t, sw2_ref[...], preferred_element_type=jnp.float32)
    spart = jnp.full((1, d), jnp.sum(sc), jnp.float32)

    rows_out = o_ref.shape[1]
    pieces = [g, spart]
    if rows_out > gpt + 1:
        pieces.append(jnp.zeros((rows_out - gpt - 1, d), jnp.float32))
    o_ref[...] = jnp.concatenate(pieces, axis=0)[None]


def _gat_all(adj, h, w, al, ar, bias, sw1, sb1, sw2,
             *, num_heads, head_dim, tile_m, npg):
    m, n, _ = adj.shape
    fin = h.shape[-1]
    d = num_heads * head_dim
    tiles = n // tile_m
    bph = tiles // 2

    flops = int(m * (2 * n * fin * d
                     + num_heads * (2 * n * head_dim + 2 * n * n * head_dim)
                     + 2 * n * d * 128))
    transcendentals = int(m * n * 130)
    bytes_accessed = int(m * (n * n * adj.dtype.itemsize + n * fin * 4))

    gpt = tile_m // npg
    rows_out = ((gpt + 1 + 7) // 8) * 8
    body = functools.partial(_gat_kernel, num_heads=num_heads,
                             head_dim=head_dim, tile_m=tile_m, bph=bph,
                             npg=npg)
    return pl.pallas_call(
        body,
        out_shape=jax.ShapeDtypeStruct((m, tiles, rows_out, d), jnp.float32),
        grid=(m, 2, bph),
        in_specs=[
            pl.BlockSpec((None, tile_m, n),
                         lambda i, hf, r: (i, hf * bph + r, 0)),
            pl.BlockSpec((None, n, fin), lambda i, hf, r: (i, 0, 0)),
            pl.BlockSpec((None, fin, d), lambda i, hf, r: (i, 0, 0)),
            pl.BlockSpec((None, num_heads, head_dim),
                         lambda i, hf, r: (i, 0, 0)),
            pl.BlockSpec((None, num_heads, head_dim),
                         lambda i, hf, r: (i, 0, 0)),
            pl.BlockSpec((None, 1, d), lambda i, hf, r: (i, 0, 0)),
            pl.BlockSpec(sw1.shape, lambda i, hf, r: (0, 0)),
            pl.BlockSpec(sb1.shape, lambda i, hf, r: (0, 0)),
            pl.BlockSpec(sw2.shape, lambda i, hf, r: (0, 0)),
        ],
        out_specs=pl.BlockSpec((None, 1, rows_out, d),
                               lambda i, hf, r: (i, hf * bph + r, 0, 0)),
        scratch_shapes=[
            pltpu.VMEM((n, d), jnp.float32),
            pltpu.VMEM((8, n), jnp.bfloat16),
            pltpu.VMEM((8, 128), jnp.float32),
            pltpu.VMEM((n, 2 * head_dim), jnp.bfloat16),
            pltpu.VMEM((n, 2 * head_dim), jnp.bfloat16),
        ],
        compiler_params=pltpu.CompilerParams(
            dimension_semantics=("parallel", "parallel", "arbitrary"),
            vmem_limit_bytes=64 * 1024 * 1024),
        cost_estimate=pl.CostEstimate(flops=flops,
                                      transcendentals=transcendentals,
                                      bytes_accessed=bytes_accessed),
    )(adj, h, w, al, ar, bias, sw1, sb1, sw2)


def _head_kernel(ga_ref, pw_ref, pb_ref, mw1_ref, mb1_ref, mw2_ref, mb2_ref,
                 mw3_ref, mb3_ref, o_ref, *, n_nodes, npg, tile_m):
    ga = ga_ref[...]
    mcount, tiles, _, d = ga.shape
    gpt = tile_m // npg

    sc = ga[:, :, gpt:gpt + 1, 0:1]
    scores = jnp.sum(sc, axis=1, keepdims=True) * (1.0 / n_nodes)
    mx = jnp.max(scores, axis=0, keepdims=True)
    e = jnp.exp(scores - mx)
    beta = e / jnp.sum(e, axis=0, keepdims=True)

    gsum = jnp.sum(ga[:, :, 0:gpt, :] * beta, axis=0)
    gm = gsum.reshape(tiles * gpt, d)

    g = (jnp.dot(gm, pw_ref[...], preferred_element_type=jnp.float32)
         + npg * pb_ref[...])
    x = jnp.maximum(jnp.dot(g, mw1_ref[...],
                            preferred_element_type=jnp.float32)
                    + mb1_ref[...], 0.0)
    x = jnp.maximum(jnp.dot(x, mw2_ref[...],
                            preferred_element_type=jnp.float32)
                    + mb2_ref[...], 0.0)
    logits = jnp.dot(x, mw3_ref[...],
                     preferred_element_type=jnp.float32) + mb3_ref[...]
    mmax = jnp.max(logits, axis=1, keepdims=True)
    p = jnp.exp(logits - mmax)
    o_ref[...] = p / jnp.sum(p, axis=1, keepdims=True)


def _head(ga, pred_w, pred_b, mlp, *, batch, npg, tile_m):
    n_nodes = batch * npg
    body = functools.partial(_head_kernel, n_nodes=n_nodes, npg=npg,
                             tile_m=tile_m)
    vspec = pl.BlockSpec(memory_space=VMEM)
    return pl.pallas_call(
        body,
        out_shape=jax.ShapeDtypeStruct((batch, 2), jnp.float32),
        in_specs=[vspec] * 9,
        out_specs=vspec,
    )(ga, pred_w, pred_b, mlp["w1"], mlp["b1"], mlp["w2"], mlp["b2"],
      mlp["w3"], mlp["b3"])


def _han(gat_w, gat_al, gat_ar, gat_bias, sem_w1, sem_b1, sem_w2,
         pred_w, pred_b, mlp, adj, h, *, num_heads, head_dim, batch, npg,
         tile_m):
    ga = _gat_all(adj, h, gat_w, gat_al, gat_ar, gat_bias,
                  sem_w1, sem_b1, sem_w2,
                  num_heads=num_heads, head_dim=head_dim, tile_m=tile_m,
                  npg=npg)
    return _head(ga, pred_w, pred_b, mlp, batch=batch, npg=npg, tile_m=tile_m)


def kernel(gat_w, gat_al, gat_ar, gat_bias, sem_w1, sem_b1, sem_w2,
           pred_w, pred_b, mlp_w1, mlp_b1, mlp_w2, mlp_b2, mlp_w3, mlp_b3,
           adj, h):
    mlp = {"w1": mlp_w1, "b1": mlp_b1, "w2": mlp_w2, "b2": mlp_b2,
           "w3": mlp_w3, "b3": mlp_b3}
    return _han(gat_w, gat_al, gat_ar, gat_bias, sem_w1, sem_b1, sem_w2,
                pred_w, pred_b, mlp, adj, h,
                num_heads=2, head_dim=8, batch=64, npg=64, tile_m=512)
